# baseline (speedup 1.0000x reference)
import functools

import jax
import jax.numpy as jnp
from jax import lax
from jax.experimental import pallas as pl
from jax.experimental.pallas import tpu as pltpu
from jax.experimental.pallas import tpu_sc as plsc

_NC = 2
_NS = 16
_NW = _NC * _NS
_CH = 128
_HCH = 2 * _CH


@functools.lru_cache(maxsize=None)
def _make_gather(batch: int, hist: int, d: int):
    bw = batch // _NW
    n_chunk = hist * bw // _HCH
    n_stream = 2 * n_chunk
    mesh = plsc.VectorSubcoreMesh(core_axis_name="c", subcore_axis_name="s")

    @functools.partial(
        pl.kernel,
        out_type=jax.ShapeDtypeStruct((hist, d, batch), jnp.float32),
        mesh=mesh,
        scratch_types=[
            pltpu.VMEM((n_stream, _CH), jnp.int32),
            pltpu.VMEM((2, _HCH, d), jnp.float32),
            pltpu.VMEM((2, d, _HCH), jnp.float32),
            pltpu.SemaphoreType.DMA((2,)),
            pltpu.SemaphoreType.DMA((2,)),
        ],
        compiler_params=pltpu.CompilerParams(use_tc_tiling_on_sc=False,
                                             needs_layout_passes=False),
    )
    def gather_k(idx_hbm, table_hbm, out_hbm, idx_v, raw_v, tb_v, gsem, wsem):
        wid = lax.axis_index("s") * _NC + lax.axis_index("c")
        b0w = wid * bw

        pltpu.sync_copy(idx_hbm.at[wid], idx_v)

        rows_vecs = [lax.iota(jnp.int32, 16) + g * 16 for g in range(16)]

        def start_gather(c, p):
            for k in range(2):
                pltpu.async_copy(table_hbm.at[idx_v.at[2 * c + k]],
                                 raw_v.at[p, pl.ds(k * _CH, _CH)],
                                 gsem.at[p])

        def wait_gather(p):
            for _ in range(2):
                pltpu.make_async_copy(table_hbm.at[idx_v.at[0]],
                                      raw_v.at[p, pl.ds(0, _CH)],
                                      gsem.at[p]).wait()

        def start_write(c, q):
            h = c // 2
            half = c - 2 * h
            dst = out_hbm.at[h, :, pl.ds(b0w + half * _HCH, _HCH)]
            pltpu.async_copy(tb_v.at[q], dst, wsem.at[q])

        def wait_write(q):
            pltpu.make_async_copy(tb_v.at[q],
                                  out_hbm.at[0, :, pl.ds(0, _HCH)],
                                  wsem.at[q]).wait()

        def transpose(p, q):
            def dbody(dd, carry):
                dvec = jnp.full((16,), dd, jnp.int32)
                for g in range(16):
                    val = plsc.load_gather(raw_v.at[p], [rows_vecs[g], dvec])
                    tb_v[q, dd, pl.ds(g * 16, 16)] = val
                return carry

            lax.fori_loop(0, d, dbody, 0)

        for p in range(2):
            start_gather(p, p)

        def outer(o, carry):
            for p in range(2):
                c = o * 2 + p
                wait_gather(p)

                @pl.when(c >= 2)
                def _():
                    wait_write(p)

                transpose(p, p)

                @pl.when(c + 2 < n_chunk)
                def _():
                    start_gather(c + 2, p)

                start_write(c, p)
            return carry

        lax.fori_loop(0, n_chunk // 2, outer, 0)

        for q in range(2):
            wait_write(q)

    return gather_k


@jax.jit
def kernel(x, table):
    batch, hist = x.shape
    d = table.shape[1]
    bw = batch // _NW
    xr = (x.reshape(_NW, 4, _CH, hist)
          .transpose(0, 3, 1, 2)
          .reshape(_NW, 4 * hist, _CH))
    out = _make_gather(batch, hist, d)(xr, table)
    return out.transpose(2, 0, 1)


# speedup vs baseline: 1.6246x; 1.6246x over previous
import functools

import jax
import jax.numpy as jnp
from jax import lax
from jax.experimental import pallas as pl
from jax.experimental.pallas import tpu as pltpu
from jax.experimental.pallas import tpu_sc as plsc

_NC = 2
_NS = 16
_NW = _NC * _NS
_CH = 128
_NBUF = 10
_AHEAD = 5


@functools.lru_cache(maxsize=None)
def _make_gather(n_chunk: int, d: int):
    b_total = _NW * n_chunk * _CH
    mesh = plsc.VectorSubcoreMesh(core_axis_name="c", subcore_axis_name="s")

    @functools.partial(
        pl.kernel,
        out_type=jax.ShapeDtypeStruct((b_total, d), jnp.float32),
        mesh=mesh,
        scratch_types=[
            pltpu.VMEM((n_chunk, _CH), jnp.int32),
            pltpu.VMEM((_NBUF, _CH, d), jnp.float32),
            pltpu.SemaphoreType.DMA((_NBUF,)),
            pltpu.SemaphoreType.DMA((_NBUF,)),
        ],
        compiler_params=pltpu.CompilerParams(use_tc_tiling_on_sc=False),
    )
    def gather_k(idx_hbm, table_hbm, out_hbm, idx_v, rows_v, gsem, wsem):
        wid = lax.axis_index("s") * _NC + lax.axis_index("c")
        base = wid * (n_chunk * _CH)

        pltpu.sync_copy(idx_hbm.at[wid], idx_v)

        def start_gather(ch, b):
            pltpu.async_copy(table_hbm.at[idx_v.at[ch]], rows_v.at[b],
                             gsem.at[b])

        def wait_gather(b):
            pltpu.make_async_copy(table_hbm.at[idx_v.at[0]], rows_v.at[b],
                                  gsem.at[b]).wait()

        def start_write(ch, b):
            pltpu.async_copy(rows_v.at[b],
                             out_hbm.at[pl.ds(base + ch * _CH, _CH)],
                             wsem.at[b])

        def wait_write(b):
            pltpu.make_async_copy(rows_v.at[b], out_hbm.at[pl.ds(0, _CH)],
                                  wsem.at[b]).wait()

        for b in range(_AHEAD):
            start_gather(b, b)

        def outer(o, carry):
            ch0 = o * _NBUF
            for b in range(_NBUF):
                ch = ch0 + b
                wait_gather(b)
                start_write(ch, b)
                nxt = ch + _AHEAD
                nb = (b + _AHEAD) % _NBUF

                @pl.when(nxt < n_chunk)
                def _():
                    @pl.when(nxt >= _NBUF)
                    def _():
                        wait_write(nb)
                    start_gather(nxt, nb)
            return carry

        lax.fori_loop(0, n_chunk // _NBUF, outer, 0)

        for b in range(_NBUF):
            wait_write(b)

    return gather_k


@jax.jit
def kernel(x, table):
    batch, hist = x.shape
    d = table.shape[1]
    b_total = batch * hist
    n_chunk = b_total // (_NW * _CH)
    assert n_chunk * _NW * _CH == b_total and n_chunk % _NBUF == 0
    xr = x.reshape(_NW, n_chunk, _CH)
    out = _make_gather(n_chunk, d)(xr, table)
    return out.reshape(batch, hist, d)
